# baseline (speedup 1.0000x reference)
import jax
import jax.numpy as jnp
from jax.experimental import pallas as pl


def _pos_body(row_ref, col_ref, out_ref):
    h, half = row_ref.shape
    w, _ = col_ref.shape
    col_t = col_ref[...].T
    row_t = row_ref[...].T
    xe = jnp.broadcast_to(col_t[:, None, :], (half, h, w)).reshape(half, h * w)
    ye = jnp.broadcast_to(row_t[:, :, None], (half, h, w)).reshape(half, h * w)
    out_ref[0] = jnp.concatenate([xe, ye], axis=0)


def kernel(x, row_w, col_w):
    n, dim, h, w = x.shape
    half = dim // 2
    out = pl.pallas_call(
        _pos_body,
        grid=(n,),
        in_specs=[
            pl.BlockSpec((h, half), lambda i: (0, 0)),
            pl.BlockSpec((w, half), lambda i: (0, 0)),
        ],
        out_specs=pl.BlockSpec((1, dim, h * w), lambda i: (i, 0, 0)),
        out_shape=jax.ShapeDtypeStruct((n, dim, h * w), x.dtype),
    )(row_w[:h], col_w[:w])
    return out.reshape(n, dim, h, w)


# speedup vs baseline: 1.4962x; 1.4962x over previous
import jax
import jax.numpy as jnp
from jax.experimental import pallas as pl
from jax.experimental.pallas import tpu as pltpu


def _pos_body(row_ref, col_ref, out_ref, tile, sem):
    h, half = row_ref.shape
    w = col_ref.shape[0]
    hw = h * w
    lane = jax.lax.broadcasted_iota(jnp.int32, (w, hw), 1)
    src = jax.lax.broadcasted_iota(jnp.int32, (w, hw), 0)
    p = (lane % w == src).astype(jnp.float32)
    lane_h = jax.lax.broadcasted_iota(jnp.int32, (h, hw), 1)
    src_h = jax.lax.broadcasted_iota(jnp.int32, (h, hw), 0)
    q = (lane_h // w == src_h).astype(jnp.float32)
    xe = jax.lax.dot_general(
        col_ref[...], p, (((0,), (0,)), ((), ())),
        preferred_element_type=jnp.float32,
    )
    ye = jax.lax.dot_general(
        row_ref[...], q, (((0,), (0,)), ((), ())),
        preferred_element_type=jnp.float32,
    )
    tile[0:half, :] = xe
    tile[half:, :] = ye
    n = out_ref.shape[0]
    for k in range(n):
        pltpu.make_async_copy(tile, out_ref.at[k], sem.at[k]).start()
    for k in range(n):
        pltpu.make_async_copy(tile, out_ref.at[k], sem.at[k]).wait()


def kernel(x, row_w, col_w):
    n, dim, h, w = x.shape
    half = dim // 2
    out = pl.pallas_call(
        _pos_body,
        in_specs=[
            pl.BlockSpec(memory_space=pltpu.VMEM),
            pl.BlockSpec(memory_space=pltpu.VMEM),
        ],
        out_specs=pl.BlockSpec(memory_space=pl.ANY),
        out_shape=jax.ShapeDtypeStruct((n, dim, h * w), jnp.float32),
        scratch_shapes=[
            pltpu.VMEM((dim, h * w), jnp.float32),
            pltpu.SemaphoreType.DMA((n,)),
        ],
    )(row_w[:h], col_w[:w])
    return out.reshape(n, dim, h, w)
